# baseline (speedup 1.0000x reference)
import functools

import jax
import jax.numpy as jnp
from jax.experimental import pallas as pl
from jax.experimental.pallas import tpu as pltpu

E = 8
R = 16
ER = E * R
TOP_K = 2
SCALING = 32.0 / 16.0


def _fused_body(x_ref, wb_ref, b_ref, ga_ref, bt_ref, out_ref):
    tm = x_ref.shape[0]
    xt = x_ref[...]
    acc = jax.lax.dot_general(
        xt, wb_ref[...], (((1,), (1,)), ((), ())),
        preferred_element_type=jnp.float32)
    hg = jax.lax.dot_general(
        xt, ga_ref[...], (((1,), (1,)), ((), ())),
        preferred_element_type=jnp.float32)
    h = hg[:, :ER]
    logits = hg[:, ER:]
    eidx = jax.lax.broadcasted_iota(jnp.int32, (tm, E), 1)
    m1 = jnp.max(logits, axis=1, keepdims=True)
    i1 = jnp.min(jnp.where(logits == m1, eidx, E), axis=1, keepdims=True)
    p2 = jnp.where(eidx == i1, -jnp.inf, logits)
    m2 = jnp.max(p2, axis=1, keepdims=True)
    i2 = jnp.min(jnp.where(p2 == m2, eidx, E), axis=1, keepdims=True)
    w1 = SCALING / (1.0 + jnp.exp(m2 - m1))
    w2 = SCALING - w1
    cw = jnp.where(eidx == i1, w1, 0.0) + jnp.where(eidx == i2, w2, 0.0)
    ex_r = jax.lax.broadcasted_iota(jnp.int32, (E, ER), 0)
    ex_c = jax.lax.broadcasted_iota(jnp.int32, (E, ER), 1)
    expand = (ex_r == ex_c // R).astype(jnp.float32)
    cwe = jax.lax.dot_general(
        cw, expand, (((1,), (0,)), ((), ())),
        preferred_element_type=jnp.float32)
    hw = h * cwe
    acc += jax.lax.dot_general(
        hw, bt_ref[...], (((1,), (1,)), ((), ())),
        preferred_element_type=jnp.float32)
    out_ref[...] = acc + b_ref[...]


@functools.partial(jax.jit, static_argnames=("tm",))
def _run(xf, W_bf, b2, GA, Bt, tm):
    T, D = xf.shape
    return pl.pallas_call(
        _fused_body,
        grid=(T // tm,),
        in_specs=[
            pl.BlockSpec((tm, D), lambda i: (i, 0)),
            pl.BlockSpec((D, D), lambda i: (0, 0)),
            pl.BlockSpec((1, D), lambda i: (0, 0)),
            pl.BlockSpec((ER + E, D), lambda i: (0, 0)),
            pl.BlockSpec((D, ER), lambda i: (0, 0)),
        ],
        out_specs=pl.BlockSpec((tm, D), lambda i: (i, 0)),
        out_shape=jax.ShapeDtypeStruct((T, D), jnp.float32),
    )(xf, W_bf, b2, GA, Bt)


def kernel(x, W_base, b_base, W_gate, lora_A, lora_B):
    b, s, d = x.shape
    xf = x.reshape(-1, d)
    A_flat = lora_A.reshape(ER, d)
    GA = jnp.concatenate([A_flat, W_gate], axis=0)
    Bt = lora_B.transpose(1, 0, 2).reshape(d, ER)
    b2 = b_base.reshape(1, d)
    out = _run(xf, W_base, b2, GA, Bt, tm=1024)
    return out.reshape(b, s, d)


# speedup vs baseline: 1.0660x; 1.0660x over previous
import functools

import jax
import jax.numpy as jnp
from jax.experimental import pallas as pl
from jax.experimental.pallas import tpu as pltpu

E = 8
R = 16
ER = E * R
TOP_K = 2
SCALING = 32.0 / 16.0


def _fused_body(x_ref, wb_ref, b_ref, ga_ref, bt_ref, out_ref):
    tm = x_ref.shape[0]
    xt = x_ref[...]
    acc = jax.lax.dot_general(
        xt, wb_ref[...], (((1,), (1,)), ((), ())),
        preferred_element_type=jnp.float32)
    h = jax.lax.dot_general(
        xt, ga_ref[:ER, :], (((1,), (1,)), ((), ())),
        preferred_element_type=jnp.float32)
    logits = jax.lax.dot_general(
        xt, ga_ref[ER:, :], (((1,), (1,)), ((), ())),
        preferred_element_type=jnp.float32)
    eidx = jax.lax.broadcasted_iota(jnp.int32, (tm, E), 1)
    m1 = jnp.max(logits, axis=1, keepdims=True)
    i1 = jnp.min(jnp.where(logits == m1, eidx, E), axis=1, keepdims=True)
    p2 = jnp.where(eidx == i1, -jnp.inf, logits)
    m2 = jnp.max(p2, axis=1, keepdims=True)
    i2 = jnp.min(jnp.where(p2 == m2, eidx, E), axis=1, keepdims=True)
    w1 = SCALING / (1.0 + jnp.exp(m2 - m1))
    w2 = SCALING - w1
    cidx = jax.lax.broadcasted_iota(jnp.int32, (tm, ER), 1)
    ec = cidx // R
    cwe = jnp.where(ec == i1, w1, 0.0) + jnp.where(ec == i2, w2, 0.0)
    hw = h * cwe
    acc += jax.lax.dot_general(
        hw, bt_ref[...], (((1,), (1,)), ((), ())),
        preferred_element_type=jnp.float32)
    out_ref[...] = acc + b_ref[...]


@functools.partial(jax.jit, static_argnames=("tm",))
def _run(xf, W_bf, b2, GA, Bt, tm):
    T, D = xf.shape
    return pl.pallas_call(
        _fused_body,
        grid=(T // tm,),
        in_specs=[
            pl.BlockSpec((tm, D), lambda i: (i, 0)),
            pl.BlockSpec((D, D), lambda i: (0, 0)),
            pl.BlockSpec((1, D), lambda i: (0, 0)),
            pl.BlockSpec((ER + E, D), lambda i: (0, 0)),
            pl.BlockSpec((D, ER), lambda i: (0, 0)),
        ],
        out_specs=pl.BlockSpec((tm, D), lambda i: (i, 0)),
        out_shape=jax.ShapeDtypeStruct((T, D), jnp.float32),
    )(xf, W_bf, b2, GA, Bt)


def kernel(x, W_base, b_base, W_gate, lora_A, lora_B):
    b, s, d = x.shape
    xf = x.reshape(-1, d)
    A_flat = lora_A.reshape(ER, d)
    GA = jnp.concatenate([A_flat, W_gate], axis=0)
    Bt = lora_B.transpose(1, 0, 2).reshape(d, ER)
    b2 = b_base.reshape(1, d)
    out = _run(xf, W_base, b2, GA, Bt, tm=1024)
    return out.reshape(b, s, d)
